# baseline (speedup 1.0000x reference)
import functools
import math

import jax
import jax.numpy as jnp
from jax import lax
from jax.experimental import pallas as pl
from jax.experimental.pallas import tpu as pltpu
from jax.experimental.pallas import tpu_sc as plsc

MODEL_DIM = 64
VOCAB = 1000000
SCALE = math.sqrt(MODEL_DIM)

NUM_CORES = 2
NUM_SUBCORES = 16
NUM_WORKERS = NUM_CORES * NUM_SUBCORES
LANES = 16

TBLK = 128
FULL_BLK = VOCAB // TBLK
TAIL = VOCAB - FULL_BLK * TBLK
NBLK = FULL_BLK + 1

CHUNK = 512
SUBGATHER = 128

_MESH = dict(core_axis_name="c", subcore_axis_name="s",
             num_cores=NUM_CORES, num_subcores=NUM_SUBCORES)


def _make_format_kernel():
    @functools.partial(
        pl.kernel,
        out_type=jax.ShapeDtypeStruct((VOCAB * MODEL_DIM,), jnp.float32),
        mesh=plsc.VectorSubcoreMesh(**_MESH),
        scratch_types=[
            pltpu.VMEM((MODEL_DIM, TBLK), jnp.float32),
            pltpu.VMEM((TBLK * MODEL_DIM,), jnp.float32),
        ],
        compiler_params=pltpu.CompilerParams(use_tc_tiling_on_sc=True,
                                             needs_layout_passes=False),
    )
    def fmt(tt_hbm, tail_hbm, out_hbm, slab_v, trans_v):
        wid = lax.axis_index("s") * NUM_CORES + lax.axis_index("c")
        nblk_w = (FULL_BLK - 1 - wid) // NUM_WORKERS + 1
        lane = lax.iota(jnp.int32, LANES)

        def blk(k, _):
            base = (wid + NUM_WORKERS * k) * TBLK
            pltpu.sync_copy(tt_hbm.at[:, pl.ds(base, TBLK)], slab_v)

            def col(i, _):
                for g in range(MODEL_DIM // LANES):
                    rows = lane + (g * LANES)
                    cols = jnp.full((LANES,), i, jnp.int32)
                    v = plsc.load_gather(slab_v, [rows, cols])
                    trans_v[pl.ds(i * MODEL_DIM + g * LANES, LANES)] = v
                return 0

            lax.fori_loop(0, TBLK, col, 0, unroll=2)
            pltpu.sync_copy(trans_v,
                            out_hbm.at[pl.ds(base * MODEL_DIM,
                                             TBLK * MODEL_DIM)])
            return 0

        lax.fori_loop(0, nblk_w, blk, 0)

        @pl.when(wid == NUM_WORKERS - 1)
        def _tail():
            pltpu.sync_copy(tail_hbm, trans_v.at[pl.ds(0, TAIL * MODEL_DIM)])
            pltpu.sync_copy(trans_v.at[pl.ds(0, TAIL * MODEL_DIM)],
                            out_hbm.at[pl.ds(FULL_BLK * TBLK * MODEL_DIM,
                                             TAIL * MODEL_DIM)])

        return None

    return fmt


def _make_gather_kernel(B: int):
    b_per_w = B // NUM_WORKERS
    steps = b_per_w // CHUNK
    n_sub = CHUNK // SUBGATHER

    @functools.partial(
        pl.kernel,
        out_type=jax.ShapeDtypeStruct((B, MODEL_DIM), jnp.float32),
        mesh=plsc.VectorSubcoreMesh(**_MESH),
        scratch_types=[
            pltpu.VMEM((CHUNK,), jnp.int32),
            pltpu.VMEM((CHUNK, MODEL_DIM), jnp.float32),
            pltpu.SemaphoreType.DMA,
        ],
        compiler_params=pltpu.CompilerParams(use_tc_tiling_on_sc=False),
    )
    def gat(idx_hbm, table_hbm, out_hbm, idx_v, rows_v, sem):
        wid = lax.axis_index("s") * NUM_CORES + lax.axis_index("c")
        base = wid * b_per_w

        def step(s, _):
            off = pl.multiple_of(base + s * CHUNK, CHUNK)
            pltpu.sync_copy(idx_hbm.at[pl.ds(off, CHUNK)], idx_v)
            copies = [
                pltpu.async_copy(
                    table_hbm.at[idx_v.at[pl.ds(q * SUBGATHER, SUBGATHER)]],
                    rows_v.at[pl.ds(q * SUBGATHER, SUBGATHER), :],
                    sem)
                for q in range(n_sub)
            ]
            for c in copies:
                c.wait()

            def scale_row(i, _):
                for g in range(MODEL_DIM // LANES):
                    sl = pl.ds(g * LANES, LANES)
                    rows_v[i, sl] = rows_v[i, sl] * SCALE
                return 0

            lax.fori_loop(0, CHUNK, scale_row, 0, unroll=4)
            pltpu.sync_copy(rows_v, out_hbm.at[pl.ds(off, CHUNK), :])
            return 0

        lax.fori_loop(0, steps, step, 0)

    return gat


def kernel(x, table):
    B = x.size
    idx = x.reshape(B).astype(jnp.int32)
    tail = table[FULL_BLK * TBLK:, :].reshape(TAIL * MODEL_DIM)
    table_r = _make_format_kernel()(table.T, tail).reshape(VOCAB, MODEL_DIM)
    out = _make_gather_kernel(B)(idx, table_r)
    return out.reshape(x.shape + (MODEL_DIM,))


# speedup vs baseline: 1.3741x; 1.3741x over previous
import functools
import math

import jax
import jax.numpy as jnp
from jax import lax
from jax.experimental import pallas as pl
from jax.experimental.pallas import tpu as pltpu
from jax.experimental.pallas import tpu_sc as plsc

MODEL_DIM = 64
VOCAB = 1000000
SCALE = math.sqrt(MODEL_DIM)

NUM_CORES = 2
NUM_SUBCORES = 16
NUM_WORKERS = NUM_CORES * NUM_SUBCORES
LANES = 16

TBLK = 128
FULL_BLK = VOCAB // TBLK
TAIL = VOCAB - FULL_BLK * TBLK
NBLK = FULL_BLK + 1

CHUNK = 512
SUBGATHER = 128

_MESH = dict(core_axis_name="c", subcore_axis_name="s",
             num_cores=NUM_CORES, num_subcores=NUM_SUBCORES)


def _make_format_kernel():
    @functools.partial(
        pl.kernel,
        out_type=jax.ShapeDtypeStruct((VOCAB * MODEL_DIM,), jnp.float32),
        mesh=plsc.VectorSubcoreMesh(**_MESH),
        scratch_types=[
            pltpu.VMEM((MODEL_DIM, TBLK), jnp.float32),
            pltpu.VMEM((TBLK * MODEL_DIM,), jnp.float32),
        ],
        compiler_params=pltpu.CompilerParams(use_tc_tiling_on_sc=True,
                                             needs_layout_passes=False),
    )
    def fmt(tt_hbm, tail_hbm, out_hbm, slab_v, trans_v):
        wid = lax.axis_index("s") * NUM_CORES + lax.axis_index("c")
        nblk_w = (FULL_BLK - 1 - wid) // NUM_WORKERS + 1
        lane = lax.iota(jnp.int32, LANES)

        def blk(k, _):
            base = (wid + NUM_WORKERS * k) * TBLK
            pltpu.sync_copy(tt_hbm.at[:, pl.ds(base, TBLK)], slab_v)

            @plsc.parallel_loop(0, TBLK, unroll=8)
            def col(i):
                for g in range(MODEL_DIM // LANES):
                    rows = lane + (g * LANES)
                    cols = jnp.full((LANES,), i, jnp.int32)
                    v = plsc.load_gather(slab_v, [rows, cols])
                    trans_v[pl.ds(i * MODEL_DIM + g * LANES, LANES)] = v
            pltpu.sync_copy(trans_v,
                            out_hbm.at[pl.ds(base * MODEL_DIM,
                                             TBLK * MODEL_DIM)])
            return 0

        lax.fori_loop(0, nblk_w, blk, 0)

        @pl.when(wid == NUM_WORKERS - 1)
        def _tail():
            pltpu.sync_copy(tail_hbm, trans_v.at[pl.ds(0, TAIL * MODEL_DIM)])
            pltpu.sync_copy(trans_v.at[pl.ds(0, TAIL * MODEL_DIM)],
                            out_hbm.at[pl.ds(FULL_BLK * TBLK * MODEL_DIM,
                                             TAIL * MODEL_DIM)])

        return None

    return fmt


def _make_gather_kernel(B: int):
    b_per_w = B // NUM_WORKERS
    steps = b_per_w // CHUNK
    n_sub = CHUNK // SUBGATHER

    @functools.partial(
        pl.kernel,
        out_type=jax.ShapeDtypeStruct((B, MODEL_DIM), jnp.float32),
        mesh=plsc.VectorSubcoreMesh(**_MESH),
        scratch_types=[
            pltpu.VMEM((CHUNK,), jnp.int32),
            pltpu.VMEM((CHUNK, MODEL_DIM), jnp.float32),
            pltpu.SemaphoreType.DMA,
        ],
        compiler_params=pltpu.CompilerParams(use_tc_tiling_on_sc=False),
    )
    def gat(idx_hbm, table_hbm, out_hbm, idx_v, rows_v, sem):
        wid = lax.axis_index("s") * NUM_CORES + lax.axis_index("c")
        base = wid * b_per_w

        def step(s, _):
            off = pl.multiple_of(base + s * CHUNK, CHUNK)
            pltpu.sync_copy(idx_hbm.at[pl.ds(off, CHUNK)], idx_v)
            copies = [
                pltpu.async_copy(
                    table_hbm.at[idx_v.at[pl.ds(q * SUBGATHER, SUBGATHER)]],
                    rows_v.at[pl.ds(q * SUBGATHER, SUBGATHER), :],
                    sem)
                for q in range(n_sub)
            ]
            for c in copies:
                c.wait()

            @plsc.parallel_loop(0, CHUNK, unroll=8)
            def scale_row(i):
                for g in range(MODEL_DIM // LANES):
                    sl = pl.ds(g * LANES, LANES)
                    rows_v[i, sl] = rows_v[i, sl] * SCALE
            pltpu.sync_copy(rows_v, out_hbm.at[pl.ds(off, CHUNK), :])
            return 0

        lax.fori_loop(0, steps, step, 0)

    return gat


def kernel(x, table):
    B = x.size
    idx = x.reshape(B).astype(jnp.int32)
    tail = table[FULL_BLK * TBLK:, :].reshape(TAIL * MODEL_DIM)
    table_r = _make_format_kernel()(table.T, tail).reshape(VOCAB, MODEL_DIM)
    out = _make_gather_kernel(B)(idx, table_r)
    return out.reshape(x.shape + (MODEL_DIM,))
